# baseline (speedup 1.0000x reference)
import functools
import math

import jax
import jax.numpy as jnp
from jax.experimental import pallas as pl
from jax.experimental.pallas import tpu as pltpu


def _router_prologue(x, ee, cap):
    N, _ = x.shape
    E = ee.shape[0]

    logits = jax.lax.dot_general(
        x, ee, (((1,), (1,)), ((), ())), preferred_element_type=jnp.float32)
    m = jnp.max(logits, axis=1, keepdims=True)
    ex = jnp.exp(logits - m)
    s = jnp.sum(ex, axis=1, keepdims=True)
    soft = ex / s

    w = jnp.max(soft, axis=1, keepdims=True)
    ecol = jax.lax.broadcasted_iota(jnp.int32, (N, E), 1)
    cand = jnp.where(soft >= w, ecol, E)
    ai = jnp.min(cand, axis=1, keepdims=True)
    oh = (ecol == ai).astype(jnp.float32)

    BLK = 256
    r_i = jax.lax.broadcasted_iota(jnp.int32, (BLK, BLK), 0)
    c_i = jax.lax.broadcasted_iota(jnp.int32, (BLK, BLK), 1)
    tri = (r_i >= c_i).astype(jnp.float32)
    cs_blocks = []
    tot_blocks = []
    for b in range(N // BLK):
        ohb = oh[b * BLK:(b + 1) * BLK, :]
        csb = jnp.dot(tri, ohb, preferred_element_type=jnp.float32)
        cs_blocks.append(csb)
        tot_blocks.append(csb[BLK - 1:BLK, :])
    off = jnp.zeros((1, E), jnp.float32)
    cnt_blocks = []
    for b in range(N // BLK):
        cnt_blocks.append(cs_blocks[b] + off)
        off = off + tot_blocks[b]
    cnt = jnp.concatenate(cnt_blocks, axis=0)
    pos = jnp.round(jnp.sum(cnt * oh, axis=1, keepdims=True)).astype(
        jnp.int32) - 1
    disp = pos < cap

    ccol = jax.lax.broadcasted_iota(jnp.int32, (N, cap), 1)
    P = jnp.where((pos == ccol) & disp, 1.0, 0.0)

    nrow = jax.lax.broadcasted_iota(
        jnp.int32, (N, 1), 0).astype(jnp.float32)
    cdims = (((0,), (0,)), ((), ()))
    hi = jax.lax.Precision.HIGHEST
    ids_f = jax.lax.dot_general(oh * nrow, P, cdims, precision=hi,
                                preferred_element_type=jnp.float32)
    valid = jax.lax.dot_general(oh, P, cdims, precision=hi,
                                preferred_element_type=jnp.float32)
    vals = jax.lax.dot_general(oh * w, P, cdims, precision=hi,
                                preferred_element_type=jnp.float32)
    ids = jnp.round(ids_f).astype(jnp.int32)
    ids = jnp.where(valid > 0.5, ids, N)

    count = jnp.sum(oh, axis=0, keepdims=True)
    colsum = jnp.sum(soft, axis=0, keepdims=True)
    loss = ((E / (N * N)) * jnp.sum(count * colsum)).reshape(1, 1)
    return ids, vals, loss


def _moe_kernel(x_ref, ee_ref, w1_ref, w2_ref, out_ref, loss_ref,
                tok_ref, acc_ref, vals_vmem_ref, ids_vmem_ref, ids_smem_ref,
                vals_smem_ref, sem_i, sem_v, *, cap, nk):
    e = pl.program_id(0)
    k = pl.program_id(1)
    N = x_ref.shape[0]

    @pl.when((e == 0) & (k == 0))
    def _prologue():
        ids, vals, loss = _router_prologue(x_ref[...], ee_ref[...], cap)
        ids_vmem_ref[...] = ids
        vals_vmem_ref[...] = vals
        loss_ref[...] = loss
        out_ref[...] = jnp.zeros_like(out_ref)
        copy_i = pltpu.make_async_copy(ids_vmem_ref, ids_smem_ref, sem_i)
        copy_v = pltpu.make_async_copy(vals_vmem_ref, vals_smem_ref, sem_v)
        copy_i.start()
        copy_v.start()
        copy_i.wait()
        copy_v.wait()

    @pl.when(k == 0)
    def _gather():
        for c in range(cap):
            tid = jnp.minimum(ids_smem_ref[e, c], N - 1)
            tok_ref[c:c + 1, :] = x_ref[pl.ds(tid, 1), :]

    tok = tok_ref[...]
    w1 = w1_ref[0]
    w2 = w2_ref[0]
    cdims = (((1,), (1,)), ((), ()))
    inter = jax.lax.dot_general(tok, w1, cdims,
                                preferred_element_type=jnp.float32)
    inter = jnp.maximum(inter, 0.0)
    part = jax.lax.dot_general(inter, w2, cdims,
                               preferred_element_type=jnp.float32)

    @pl.when(k == 0)
    def _acc0():
        acc_ref[...] = part

    @pl.when(k > 0)
    def _accn():
        acc_ref[...] += part

    @pl.when(k == nk - 1)
    def _scatter():
        rows = acc_ref[...]
        for c in range(cap):
            tid = ids_smem_ref[e, c]

            @pl.when(tid < N)
            def _store():
                out_ref[pl.ds(tid, 1), :] = rows[c:c + 1, :] * vals_smem_ref[e, c]


def kernel(x, expert_embeddings, first_linear, second_linear):
    B, S, H = x.shape
    E, I, _ = first_linear.shape
    N = B * S
    cap = math.ceil(N / E)

    xf = x.reshape(N, H)

    TI = I // 3
    nk = I // TI
    out, loss = pl.pallas_call(
        functools.partial(_moe_kernel, cap=cap, nk=nk),
        grid=(E, nk),
        in_specs=[
            pl.BlockSpec((N, H), lambda e, k: (0, 0)),
            pl.BlockSpec((E, H), lambda e, k: (0, 0)),
            pl.BlockSpec((1, TI, H), lambda e, k: (e, k, 0)),
            pl.BlockSpec((1, H, TI), lambda e, k: (e, 0, k)),
        ],
        out_specs=[
            pl.BlockSpec((N, H), lambda e, k: (0, 0)),
            pl.BlockSpec((1, 1), lambda e, k: (0, 0)),
        ],
        out_shape=[
            jax.ShapeDtypeStruct((N, H), jnp.float32),
            jax.ShapeDtypeStruct((1, 1), jnp.float32),
        ],
        scratch_shapes=[
            pltpu.VMEM((cap, H), jnp.float32),
            pltpu.VMEM((cap, H), jnp.float32),
            pltpu.VMEM((E, cap), jnp.float32),
            pltpu.VMEM((E, cap), jnp.int32),
            pltpu.SMEM((E, cap), jnp.int32),
            pltpu.SMEM((E, cap), jnp.float32),
            pltpu.SemaphoreType.DMA,
            pltpu.SemaphoreType.DMA,
        ],
    )(xf, expert_embeddings, first_linear, second_linear)

    return out.reshape(B, S, H), loss[0, 0]


# speedup vs baseline: 1.1640x; 1.1640x over previous
import functools
import math

import jax
import jax.numpy as jnp
from jax.experimental import pallas as pl
from jax.experimental.pallas import tpu as pltpu


def _router_prologue(x, ee, cap):
    N, _ = x.shape
    E = ee.shape[0]

    logits = jax.lax.dot_general(
        x, ee, (((1,), (1,)), ((), ())), preferred_element_type=jnp.float32)
    m = jnp.max(logits, axis=1, keepdims=True)
    ex = jnp.exp(logits - m)
    s = jnp.sum(ex, axis=1, keepdims=True)
    soft = ex / s

    w = jnp.max(soft, axis=1, keepdims=True)
    ecol = jax.lax.broadcasted_iota(jnp.int32, (N, E), 1)
    cand = jnp.where(soft >= w, ecol, E)
    ai = jnp.min(cand, axis=1, keepdims=True)
    oh = (ecol == ai).astype(jnp.float32)

    BLK = 256
    r_i = jax.lax.broadcasted_iota(jnp.int32, (BLK, BLK), 0)
    c_i = jax.lax.broadcasted_iota(jnp.int32, (BLK, BLK), 1)
    tri = (r_i >= c_i).astype(jnp.float32)
    cs_blocks = []
    tot_blocks = []
    for b in range(N // BLK):
        ohb = oh[b * BLK:(b + 1) * BLK, :]
        csb = jnp.dot(tri, ohb, preferred_element_type=jnp.float32)
        cs_blocks.append(csb)
        tot_blocks.append(csb[BLK - 1:BLK, :])
    off = jnp.zeros((1, E), jnp.float32)
    cnt_blocks = []
    for b in range(N // BLK):
        cnt_blocks.append(cs_blocks[b] + off)
        off = off + tot_blocks[b]
    cnt = jnp.concatenate(cnt_blocks, axis=0)
    pos = jnp.round(jnp.sum(cnt * oh, axis=1, keepdims=True)).astype(
        jnp.int32) - 1
    disp = pos < cap

    ccol = jax.lax.broadcasted_iota(jnp.int32, (N, cap), 1)
    P = jnp.where((pos == ccol) & disp, 1.0, 0.0)

    nrow = jax.lax.broadcasted_iota(jnp.int32, (N, 1), 0)
    nq = (nrow // 8).astype(jnp.float32)
    nr = (nrow % 8).astype(jnp.float32)
    cdims = (((0,), (0,)), ((), ()))
    hi = jax.lax.Precision.HIGHEST
    idq = jax.lax.dot_general(oh * nq, P, cdims,
                              preferred_element_type=jnp.float32)
    idr = jax.lax.dot_general(oh * nr, P, cdims,
                              preferred_element_type=jnp.float32)
    valid = jax.lax.dot_general(oh, P, cdims,
                                preferred_element_type=jnp.float32)
    vals = jax.lax.dot_general(oh * w, P, cdims, precision=hi,
                                preferred_element_type=jnp.float32)
    ids = (jnp.round(idq).astype(jnp.int32) * 8
           + jnp.round(idr).astype(jnp.int32))
    ids = jnp.where(valid > 0.5, ids, N)

    count = jnp.sum(oh, axis=0, keepdims=True)
    colsum = jnp.sum(soft, axis=0, keepdims=True)
    loss = ((E / (N * N)) * jnp.sum(count * colsum)).reshape(1, 1)
    return ids, vals, loss


def _moe_kernel(x_ref, ee_ref, w1_ref, w2_ref, out_ref, loss_ref,
                tok_ref, vals_vmem_ref, ids_vmem_ref, ids_smem_ref,
                vals_smem_ref, sem_i, sem_v, *, cap):
    e = pl.program_id(0)
    N = x_ref.shape[0]

    @pl.when(e == 0)
    def _prologue():
        ids, vals, loss = _router_prologue(x_ref[...], ee_ref[...], cap)
        ids_vmem_ref[...] = ids
        vals_vmem_ref[...] = vals
        loss_ref[...] = loss
        out_ref[...] = jnp.zeros_like(out_ref)
        copy_i = pltpu.make_async_copy(ids_vmem_ref, ids_smem_ref, sem_i)
        copy_v = pltpu.make_async_copy(vals_vmem_ref, vals_smem_ref, sem_v)
        copy_i.start()
        copy_v.start()
        copy_i.wait()
        copy_v.wait()

    for c in range(cap):
        tid = jnp.minimum(ids_smem_ref[e, c], N - 1)
        tok_ref[c:c + 1, :] = x_ref[pl.ds(tid, 1), :]

    tok = tok_ref[...]
    w1 = w1_ref[0]
    w2 = w2_ref[0]
    cdims = (((1,), (1,)), ((), ()))
    inter = jax.lax.dot_general(tok, w1, cdims,
                                preferred_element_type=jnp.float32)
    inter = jnp.maximum(inter, 0.0)
    rows = jax.lax.dot_general(inter, w2, cdims,
                               preferred_element_type=jnp.float32)

    for c in range(cap):
        tid = ids_smem_ref[e, c]

        @pl.when(tid < N)
        def _store():
            out_ref[pl.ds(tid, 1), :] = rows[c:c + 1, :] * vals_smem_ref[e, c]


def kernel(x, expert_embeddings, first_linear, second_linear):
    B, S, H = x.shape
    E, I, _ = first_linear.shape
    N = B * S
    cap = math.ceil(N / E)

    xf = x.reshape(N, H)

    out, loss = pl.pallas_call(
        functools.partial(_moe_kernel, cap=cap),
        grid=(E,),
        in_specs=[
            pl.BlockSpec((N, H), lambda e: (0, 0)),
            pl.BlockSpec((E, H), lambda e: (0, 0)),
            pl.BlockSpec((1, I, H), lambda e: (e, 0, 0)),
            pl.BlockSpec((1, H, I), lambda e: (e, 0, 0)),
        ],
        out_specs=[
            pl.BlockSpec((N, H), lambda e: (0, 0)),
            pl.BlockSpec((1, 1), lambda e: (0, 0)),
        ],
        out_shape=[
            jax.ShapeDtypeStruct((N, H), jnp.float32),
            jax.ShapeDtypeStruct((1, 1), jnp.float32),
        ],
        scratch_shapes=[
            pltpu.VMEM((cap, H), jnp.float32),
            pltpu.VMEM((E, cap), jnp.float32),
            pltpu.VMEM((E, cap), jnp.int32),
            pltpu.SMEM((E, cap), jnp.int32),
            pltpu.SMEM((E, cap), jnp.float32),
            pltpu.SemaphoreType.DMA,
            pltpu.SemaphoreType.DMA,
        ],
    )(xf, expert_embeddings, first_linear, second_linear)

    return out.reshape(B, S, H), loss[0, 0]
